# baseline (speedup 1.0000x reference)
import functools

import jax
import jax.numpy as jnp
from jax import lax
from jax.experimental import pallas as pl
from jax.experimental.pallas import tpu as pltpu
from jax.experimental.pallas import tpu_sc as plsc

B = 4096
SEQ = 50
D = 128
BN_EPS = 1e-3
LN_EPS = 1e-3

NC = 2
NS = 16
NW = NC * NS
BPW = B // NW
CHUNK_IDX = 100
NBUF = 4
UNROLL = 2
ROWS_PER_CHUNK = CHUNK_IDX // SEQ
NCHUNK = BPW // ROWS_PER_CHUNK
NVEC = D // 16


def _sc_pool_body(xr_hbm, table_hbm, out_hbm, idx_s, idx_v, rows_v, out_v,
                  *sems):
    wid = lax.axis_index("s") * NC + lax.axis_index("c")

    pltpu.sync_copy(xr_hbm.at[pl.ds(wid * BPW, BPW)], idx_s)

    def rbody(rr, carry):
        for half, (r, cb) in enumerate(((2 * rr, 0), (2 * rr + 1, SEQ))):
            for off in (0, 16, 32, SEQ - 16):
                idx_v[rr, pl.ds(cb + off, 16)] = idx_s[r, pl.ds(off, 16)]
        return carry

    lax.fori_loop(0, NCHUNK, rbody, 0)

    bufs = [rows_v.at[k] for k in range(NBUF)]

    for k in range(NBUF):
        pltpu.async_copy(table_hbm.at[idx_v.at[k]], bufs[k], sems[k])

    def accumulate(buf, out_row0):
        def jbody(ju, acc):
            j = UNROLL * ju
            for u in range(UNROLL):
                jj = j + u
                acc = tuple(acc[d] + buf[jj, pl.ds(16 * d, 16)]
                            for d in range(NVEC)) + tuple(
                    acc[NVEC + d] + buf[SEQ + jj, pl.ds(16 * d, 16)]
                    for d in range(NVEC))
            return acc

        zero = tuple(jnp.zeros((16,), jnp.float32) for _ in range(2 * NVEC))
        acc = lax.fori_loop(0, SEQ // UNROLL, jbody, zero)
        for d in range(NVEC):
            out_v[out_row0, pl.ds(16 * d, 16)] = acc[d]
            out_v[out_row0 + 1, pl.ds(16 * d, 16)] = acc[NVEC + d]

    def ibody(i, carry):
        g = NBUF * i
        for k in range(NBUF):
            pltpu.make_async_copy(
                table_hbm.at[idx_v.at[g + k]], bufs[k], sems[k]).wait()
            accumulate(bufs[k], 2 * (g + k))

            @pl.when(i < NCHUNK // NBUF - 1)
            def _():
                pltpu.async_copy(
                    table_hbm.at[idx_v.at[g + k + NBUF]], bufs[k], sems[k])

        return carry

    lax.fori_loop(0, NCHUNK // NBUF, ibody, 0)

    pltpu.sync_copy(out_v, out_hbm.at[pl.ds(wid * BPW, BPW)])


_sc_pool = functools.partial(
    pl.kernel,
    out_type=jax.ShapeDtypeStruct((B, D), jnp.float32),
    mesh=plsc.VectorSubcoreMesh(core_axis_name="c", subcore_axis_name="s"),
    scratch_types=[
        pltpu.VMEM((BPW, SEQ), jnp.int32),
        pltpu.VMEM((NCHUNK, CHUNK_IDX), jnp.int32),
        pltpu.VMEM((NBUF, CHUNK_IDX, D), jnp.float32),
        pltpu.VMEM((BPW, D), jnp.float32),
    ] + [pltpu.SemaphoreType.DMA] * NBUF,
)(_sc_pool_body)


def _tc_dense_body(p_ref, w_ref, b_ref, bg_ref, bb_ref, bm_ref, bv_ref,
                   lg_ref, lb_ref, o_ref):
    x = p_ref[...] * (1.0 / SEQ)
    h = jnp.dot(x, w_ref[...], preferred_element_type=jnp.float32) + b_ref[...]
    bn_scale = bg_ref[...] * lax.rsqrt(bv_ref[...] + BN_EPS)
    h = (h - bm_ref[...]) * bn_scale + bb_ref[...]
    mu = jnp.mean(h, axis=1, keepdims=True)
    hc = h - mu
    var = jnp.mean(hc * hc, axis=1, keepdims=True)
    o_ref[...] = hc * lax.rsqrt(var + LN_EPS) * lg_ref[...] + lb_ref[...]


def kernel(x, table, W, b, bn_gamma, bn_beta, bn_mean, bn_var, ln_gamma,
           ln_beta):
    assert x.shape == (B, SEQ) and table.shape[1] == D

    pooled_sum = _sc_pool(x.astype(jnp.int32), table)

    vec = lambda v: v.astype(jnp.float32).reshape(1, D)
    blk = 2048
    vspec = pl.BlockSpec((1, D), lambda i: (0, 0))
    out = pl.pallas_call(
        _tc_dense_body,
        grid=(B // blk,),
        in_specs=[pl.BlockSpec((blk, D), lambda i: (i, 0)),
                  pl.BlockSpec((D, D), lambda i: (0, 0))] + [vspec] * 7,
        out_specs=pl.BlockSpec((blk, D), lambda i: (i, 0)),
        out_shape=jax.ShapeDtypeStruct((B, D), jnp.float32),
    )(pooled_sum, W, vec(b), vec(bn_gamma), vec(bn_beta), vec(bn_mean),
      vec(bn_var), vec(ln_gamma), vec(ln_beta))
    return out


# speedup vs baseline: 1.0366x; 1.0366x over previous
import functools

import jax
import jax.numpy as jnp
from jax import lax
from jax.experimental import pallas as pl
from jax.experimental.pallas import tpu as pltpu
from jax.experimental.pallas import tpu_sc as plsc

B = 4096
SEQ = 50
D = 128
BN_EPS = 1e-3
LN_EPS = 1e-3

NC = 2
NS = 16
NW = NC * NS
BPW = B // NW
CHUNK_IDX = 100
NBUF = 4
UNROLL = 2
ROWS_PER_CHUNK = CHUNK_IDX // SEQ
NCHUNK = BPW // ROWS_PER_CHUNK
NVEC = D // 16


def _sc_pool_body(xr_hbm, table_hbm, out_hbm, idx_v, rows_v, out_v, *sems):
    wid = lax.axis_index("s") * NC + lax.axis_index("c")

    pltpu.sync_copy(xr_hbm.at[pl.ds(wid * NCHUNK, NCHUNK)], idx_v)

    bufs = [rows_v.at[k] for k in range(NBUF)]

    for k in range(NBUF):
        pltpu.async_copy(table_hbm.at[idx_v.at[k]], bufs[k], sems[k])

    def accumulate(buf, out_row0):
        def jbody(ju, acc):
            j = UNROLL * ju
            for u in range(UNROLL):
                jj = j + u
                acc = tuple(acc[d] + buf[jj, pl.ds(16 * d, 16)]
                            for d in range(NVEC)) + tuple(
                    acc[NVEC + d] + buf[SEQ + jj, pl.ds(16 * d, 16)]
                    for d in range(NVEC))
            return acc

        zero = tuple(jnp.zeros((16,), jnp.float32) for _ in range(2 * NVEC))
        acc = lax.fori_loop(0, SEQ // UNROLL, jbody, zero)
        for d in range(NVEC):
            out_v[out_row0, pl.ds(16 * d, 16)] = acc[d]
            out_v[out_row0 + 1, pl.ds(16 * d, 16)] = acc[NVEC + d]

    def ibody(i, carry):
        g = NBUF * i
        for k in range(NBUF):
            pltpu.make_async_copy(
                table_hbm.at[idx_v.at[g + k]], bufs[k], sems[k]).wait()
            accumulate(bufs[k], 2 * (g + k))

            @pl.when(i < NCHUNK // NBUF - 1)
            def _():
                pltpu.async_copy(
                    table_hbm.at[idx_v.at[g + k + NBUF]], bufs[k], sems[k])

        return carry

    lax.fori_loop(0, NCHUNK // NBUF, ibody, 0)

    pltpu.sync_copy(out_v, out_hbm.at[pl.ds(wid * BPW, BPW)])


_sc_pool = functools.partial(
    pl.kernel,
    out_type=jax.ShapeDtypeStruct((B, D), jnp.float32),
    mesh=plsc.VectorSubcoreMesh(core_axis_name="c", subcore_axis_name="s"),
    scratch_types=[
        pltpu.VMEM((NCHUNK, CHUNK_IDX), jnp.int32),
        pltpu.VMEM((NBUF, CHUNK_IDX, D), jnp.float32),
        pltpu.VMEM((BPW, D), jnp.float32),
    ] + [pltpu.SemaphoreType.DMA] * NBUF,
)(_sc_pool_body)


def _tc_dense_body(p_ref, w_ref, b_ref, bg_ref, bb_ref, bm_ref, bv_ref,
                   lg_ref, lb_ref, o_ref):
    x = p_ref[...] * (1.0 / SEQ)
    h = jnp.dot(x, w_ref[...], preferred_element_type=jnp.float32) + b_ref[...]
    bn_scale = bg_ref[...] * lax.rsqrt(bv_ref[...] + BN_EPS)
    h = (h - bm_ref[...]) * bn_scale + bb_ref[...]
    mu = jnp.mean(h, axis=1, keepdims=True)
    hc = h - mu
    var = jnp.mean(hc * hc, axis=1, keepdims=True)
    o_ref[...] = hc * lax.rsqrt(var + LN_EPS) * lg_ref[...] + lb_ref[...]


def kernel(x, table, W, b, bn_gamma, bn_beta, bn_mean, bn_var, ln_gamma,
           ln_beta):
    assert x.shape == (B, SEQ) and table.shape[1] == D

    xr = x.astype(jnp.int32).reshape(B * SEQ // CHUNK_IDX, CHUNK_IDX)
    pooled_sum = _sc_pool(xr, table)

    vec = lambda v: v.astype(jnp.float32).reshape(1, D)
    blk = 2048
    vspec = pl.BlockSpec((1, D), lambda i: (0, 0))
    out = pl.pallas_call(
        _tc_dense_body,
        grid=(B // blk,),
        in_specs=[pl.BlockSpec((blk, D), lambda i: (i, 0)),
                  pl.BlockSpec((D, D), lambda i: (0, 0))] + [vspec] * 7,
        out_specs=pl.BlockSpec((blk, D), lambda i: (i, 0)),
        out_shape=jax.ShapeDtypeStruct((B, D), jnp.float32),
    )(pooled_sum, W, vec(b), vec(bn_gamma), vec(bn_beta), vec(bn_mean),
      vec(bn_var), vec(ln_gamma), vec(ln_beta))
    return out
